# Initial kernel scaffold; baseline (speedup 1.0000x reference)
import functools

import jax
import jax.numpy as jnp
from jax import lax
from jax.experimental import pallas as pl
from jax.experimental.pallas import tpu as pltpu
from jax.experimental.pallas import tpu_sc as plsc

N_NODES = 10000
N_EDGES = 320000
D = 128

NUM_CORES = 2
NUM_SUBCORES = 16
NW = NUM_CORES * NUM_SUBCORES
E_PER_W = N_EDGES // NW
CHUNK = 80
N_CHUNKS = E_PER_W // CHUNK
ROWS_PER_TILE = N_NODES // NUM_SUBCORES
ZROWS = 125


def _sc_agg(src_emb, edge_emb, src_idx, dst_idx):
    mesh = plsc.VectorSubcoreMesh(core_axis_name="c", subcore_axis_name="s")

    @functools.partial(
        pl.kernel,
        mesh=mesh,
        out_type=[
            jax.ShapeDtypeStruct((NUM_CORES, N_NODES, D), jnp.float32),
            jax.ShapeDtypeStruct((NUM_CORES, N_NODES, 16), jnp.float32),
        ],
        scratch_types=[
            pltpu.VMEM((CHUNK,), jnp.int32),
            pltpu.VMEM((CHUNK,), jnp.int32),
            pltpu.VMEM((CHUNK, D), jnp.float32),
            pltpu.VMEM((CHUNK, D), jnp.float32),
            pltpu.VMEM((CHUNK, 16), jnp.float32),
            pltpu.VMEM((ZROWS, D), jnp.float32),
            pltpu.VMEM((ROWS_PER_TILE, 16), jnp.float32),
            pltpu.VMEM_SHARED((N_NODES, D), jnp.float32),
            pltpu.VMEM_SHARED((N_NODES, 16), jnp.float32),
            pltpu.SemaphoreType.DMA,
        ],
    )
    def ker(src_hbm, edge_hbm, sidx_hbm, didx_hbm, psum_hbm, pdeg_hbm,
            sidx_v, didx_v, srows, erows, ones_v, zbuf, zdeg, ssum, sdeg,
            sem):
        cid = lax.axis_index("c")
        sid = lax.axis_index("s")
        wid = cid * NUM_SUBCORES + sid

        zvec = jnp.zeros((16,), jnp.float32)
        onevec = jnp.where(lax.iota(jnp.int32, 16) == 0, 1.0, 0.0)

        def zero_body(j, _):
            for k in range(D // 16):
                zbuf[j, pl.ds(k * 16, 16)] = zvec
            return 0

        lax.fori_loop(0, ZROWS, zero_body, 0)

        def zdeg_body(j, _):
            zdeg[j, pl.ds(0, 16)] = zvec
            return 0

        lax.fori_loop(0, ROWS_PER_TILE, zdeg_body, 0)

        def ones_body(j, _):
            ones_v[j, pl.ds(0, 16)] = onevec
            return 0

        lax.fori_loop(0, CHUNK, ones_body, 0)

        row0 = sid * ROWS_PER_TILE
        for i in range(ROWS_PER_TILE // ZROWS):
            pltpu.sync_copy(zbuf, ssum.at[pl.ds(row0 + i * ZROWS, ZROWS)])
        pltpu.sync_copy(zdeg, sdeg.at[pl.ds(row0, ROWS_PER_TILE)])
        plsc.subcore_barrier()

        def chunk_body(c, _):
            base = wid * E_PER_W + c * CHUNK
            pltpu.sync_copy(sidx_hbm.at[pl.ds(base, CHUNK)], sidx_v)
            pltpu.sync_copy(didx_hbm.at[pl.ds(base, CHUNK)], didx_v)
            pltpu.async_copy(src_hbm.at[sidx_v], srows, sem).wait()
            pltpu.sync_copy(edge_hbm.at[pl.ds(base, CHUNK)], erows)

            def add_body(j, _):
                for k in range(D // 16):
                    sl = pl.ds(k * 16, 16)
                    erows[j, sl] = erows[j, sl] + srows[j, sl]
                return 0

            lax.fori_loop(0, CHUNK, add_body, 0)

            pltpu.sync_copy(erows, ssum.at[didx_v], add=True)
            pltpu.sync_copy(ones_v, sdeg.at[didx_v], add=True)
            return 0

        lax.fori_loop(0, N_CHUNKS, chunk_body, 0)
        plsc.subcore_barrier()

        pltpu.sync_copy(ssum.at[pl.ds(row0, ROWS_PER_TILE)],
                        psum_hbm.at[cid, pl.ds(row0, ROWS_PER_TILE)])
        pltpu.sync_copy(sdeg.at[pl.ds(row0, ROWS_PER_TILE)],
                        pdeg_hbm.at[cid, pl.ds(row0, ROWS_PER_TILE)])

    return ker(src_emb, edge_emb, src_idx, dst_idx)


def _combine_body(psum_ref, pdeg_ref, out_ref):
    s = psum_ref[0] + psum_ref[1]
    d = pdeg_ref[0, :, 0] + pdeg_ref[1, :, 0]
    out_ref[...] = s / jnp.maximum(d, 1.0)[:, None]


_RB = 1000


def _combine(psum, pdeg):
    return pl.pallas_call(
        _combine_body,
        grid=(N_NODES // _RB,),
        in_specs=[
            pl.BlockSpec((NUM_CORES, _RB, D), lambda i: (0, i, 0)),
            pl.BlockSpec((NUM_CORES, _RB, 16), lambda i: (0, i, 0)),
        ],
        out_specs=pl.BlockSpec((_RB, D), lambda i: (i, 0)),
        out_shape=jax.ShapeDtypeStruct((N_NODES, D), jnp.float32),
    )(psum, pdeg)


@jax.jit
def kernel(src_embedding, edge_embedding, edge_index):
    src_idx = edge_index[0].astype(jnp.int32)
    dst_idx = edge_index[1].astype(jnp.int32)
    psum, pdeg = _sc_agg(src_embedding, edge_embedding, src_idx, dst_idx)
    return _combine(psum, pdeg)


# speedup vs baseline: 1.2883x; 1.2883x over previous
import functools

import jax
import jax.numpy as jnp
from jax import lax
from jax.experimental import pallas as pl
from jax.experimental.pallas import tpu as pltpu
from jax.experimental.pallas import tpu_sc as plsc

N_NODES = 10000
N_EDGES = 320000
D = 128
DH = D // 2

NUM_CORES = 2
NUM_SUBCORES = 16
E_PER_TILE = N_EDGES // NUM_SUBCORES
CHUNK = 80
N_CHUNKS = E_PER_TILE // CHUNK
ROWS_PER_TILE = 624
TAIL_ROWS = N_NODES - NUM_SUBCORES * ROWS_PER_TILE
ZROWS = 104


def _sc_agg(src0, src1, edge3, src_idx, dst_idx):
    mesh = plsc.VectorSubcoreMesh(core_axis_name="c", subcore_axis_name="s")

    @functools.partial(
        pl.kernel,
        mesh=mesh,
        out_type=[
            jax.ShapeDtypeStruct((NUM_CORES, N_NODES, DH), jnp.float32),
            jax.ShapeDtypeStruct((NUM_CORES, N_NODES, 16), jnp.float32),
        ],
        scratch_types=[
            pltpu.VMEM((CHUNK,), jnp.int32),
            pltpu.VMEM((CHUNK,), jnp.int32),
            pltpu.VMEM((CHUNK, DH), jnp.float32),
            pltpu.VMEM((CHUNK, DH), jnp.float32),
            pltpu.VMEM((CHUNK, 16), jnp.float32),
            pltpu.VMEM((ZROWS, DH), jnp.float32),
            pltpu.VMEM((ROWS_PER_TILE, 16), jnp.float32),
            pltpu.VMEM_SHARED((N_NODES, DH), jnp.float32),
            pltpu.VMEM_SHARED((N_NODES, 16), jnp.float32),
            pltpu.SemaphoreType.DMA,
        ],
        compiler_params=pltpu.CompilerParams(use_tc_tiling_on_sc=False),
    )
    def ker(src0_hbm, src1_hbm, edge_hbm, sidx_hbm, didx_hbm,
            psum_hbm, pdeg_hbm,
            sidx_v, didx_v, srows, erows, ones_v, zbuf, zdeg, ssum, sdeg,
            sem):
        cid = lax.axis_index("c")
        sid = lax.axis_index("s")

        zvec = jnp.zeros((16,), jnp.float32)
        onevec = jnp.where(lax.iota(jnp.int32, 16) == 0, 1.0, 0.0)

        def zero_body(j, _):
            for k in range(DH // 16):
                zbuf[j, pl.ds(k * 16, 16)] = zvec
            return 0

        lax.fori_loop(0, ZROWS, zero_body, 0)

        def zdeg_body(j, _):
            zdeg[j, pl.ds(0, 16)] = zvec
            return 0

        lax.fori_loop(0, ROWS_PER_TILE, zdeg_body, 0)

        def ones_body(j, _):
            ones_v[j, pl.ds(0, 16)] = onevec
            return 0

        lax.fori_loop(0, CHUNK, ones_body, 0)

        row0 = sid * ROWS_PER_TILE
        for i in range(ROWS_PER_TILE // ZROWS):
            pltpu.sync_copy(zbuf, ssum.at[pl.ds(row0 + i * ZROWS, ZROWS)])
        pltpu.sync_copy(zdeg, sdeg.at[pl.ds(row0, ROWS_PER_TILE)])

        tail0 = NUM_SUBCORES * ROWS_PER_TILE

        @pl.when(sid == NUM_SUBCORES - 1)
        def _zero_tail():
            pltpu.sync_copy(zbuf.at[pl.ds(0, TAIL_ROWS)],
                            ssum.at[pl.ds(tail0, TAIL_ROWS)])
            pltpu.sync_copy(zdeg.at[pl.ds(0, TAIL_ROWS)],
                            sdeg.at[pl.ds(tail0, TAIL_ROWS)])

        plsc.subcore_barrier()

        def chunk_body(c, _):
            base = sid * E_PER_TILE + c * CHUNK
            pltpu.sync_copy(sidx_hbm.at[pl.ds(base, CHUNK)], sidx_v)
            pltpu.sync_copy(didx_hbm.at[pl.ds(base, CHUNK)], didx_v)

            @pl.when(cid == 0)
            def _gather0():
                pltpu.async_copy(src0_hbm.at[sidx_v], srows, sem).wait()

            @pl.when(cid == 1)
            def _gather1():
                pltpu.async_copy(src1_hbm.at[sidx_v], srows, sem).wait()

            pltpu.sync_copy(edge_hbm.at[pl.ds(base, CHUNK), cid], erows)

            def add_body(j, _):
                for k in range(DH // 16):
                    sl = pl.ds(k * 16, 16)
                    plsc.addupdate(erows.at[j, sl], srows[j, sl])
                return 0

            lax.fori_loop(0, CHUNK, add_body, 0)

            pltpu.sync_copy(erows, ssum.at[didx_v], add=True)

            @pl.when((c < N_CHUNKS // 2) == (cid == 0))
            def _deg():
                pltpu.sync_copy(ones_v, sdeg.at[didx_v], add=True)

            return 0

        lax.fori_loop(0, N_CHUNKS, chunk_body, 0)
        plsc.subcore_barrier()

        pltpu.sync_copy(ssum.at[pl.ds(row0, ROWS_PER_TILE)],
                        psum_hbm.at[cid, pl.ds(row0, ROWS_PER_TILE)])
        pltpu.sync_copy(sdeg.at[pl.ds(row0, ROWS_PER_TILE)],
                        pdeg_hbm.at[cid, pl.ds(row0, ROWS_PER_TILE)])

        @pl.when(sid == NUM_SUBCORES - 1)
        def _write_tail():
            pltpu.sync_copy(ssum.at[pl.ds(tail0, TAIL_ROWS)],
                            psum_hbm.at[cid, pl.ds(tail0, TAIL_ROWS)])
            pltpu.sync_copy(sdeg.at[pl.ds(tail0, TAIL_ROWS)],
                            pdeg_hbm.at[cid, pl.ds(tail0, TAIL_ROWS)])

    return ker(src0, src1, edge3, src_idx, dst_idx)


def _combine_body(psum_ref, pdeg_ref, out_ref):
    s = jnp.concatenate([psum_ref[0], psum_ref[1]], axis=-1)
    d = pdeg_ref[0, :, :1] + pdeg_ref[1, :, :1]
    out_ref[...] = s / jnp.maximum(d, 1.0)


_RB = 1000


def _combine(psum, pdeg):
    return pl.pallas_call(
        _combine_body,
        grid=(N_NODES // _RB,),
        in_specs=[
            pl.BlockSpec((NUM_CORES, _RB, DH), lambda i: (0, i, 0)),
            pl.BlockSpec((NUM_CORES, _RB, 16), lambda i: (0, i, 0)),
        ],
        out_specs=pl.BlockSpec((_RB, D), lambda i: (i, 0)),
        out_shape=jax.ShapeDtypeStruct((N_NODES, D), jnp.float32),
    )(psum, pdeg)


@jax.jit
def kernel(src_embedding, edge_embedding, edge_index):
    src_idx = edge_index[0].astype(jnp.int32)
    dst_idx = edge_index[1].astype(jnp.int32)
    src0 = src_embedding[:, :DH]
    src1 = src_embedding[:, DH:]
    edge3 = edge_embedding.reshape(N_EDGES, NUM_CORES, DH)
    psum, pdeg = _sc_agg(src0, src1, edge3, src_idx, dst_idx)
    return _combine(psum, pdeg)


# speedup vs baseline: 1.9188x; 1.4894x over previous
import functools

import jax
import jax.numpy as jnp
from jax import lax
from jax.experimental import pallas as pl
from jax.experimental.pallas import tpu as pltpu
from jax.experimental.pallas import tpu_sc as plsc

N_NODES = 10000
N_EDGES = 320000
D = 128
DH = D // 2

NUM_CORES = 2
NUM_SUBCORES = 16
E_PER_TILE = N_EDGES // NUM_SUBCORES
CHUNK = 40
N_CHUNKS = E_PER_TILE // CHUNK
NBUF = 5
LA = 3
ROWS_PER_TILE = 624
TAIL_ROWS = N_NODES - NUM_SUBCORES * ROWS_PER_TILE


def _sc_agg(src0, src1, edge3, src_idx, dst_idx, z64, z16):
    mesh = plsc.VectorSubcoreMesh(core_axis_name="c", subcore_axis_name="s")

    scratch = (
        [pltpu.VMEM((1, CHUNK), jnp.int32)] * NBUF
        + [pltpu.VMEM((1, CHUNK), jnp.int32)] * NBUF
        + [pltpu.VMEM((CHUNK, DH), jnp.float32)] * NBUF
        + [pltpu.VMEM((CHUNK, DH), jnp.float32)] * NBUF
        + [
            pltpu.VMEM((CHUNK, 16), jnp.float32),
            pltpu.VMEM_SHARED((N_NODES, DH), jnp.float32),
            pltpu.VMEM_SHARED((N_NODES, 16), jnp.float32),
        ]
        + [pltpu.SemaphoreType.DMA] * (3 * NBUF)
    )

    @functools.partial(
        pl.kernel,
        mesh=mesh,
        out_type=[
            jax.ShapeDtypeStruct((NUM_CORES, N_NODES, DH), jnp.float32),
            jax.ShapeDtypeStruct((NUM_CORES, N_NODES, 16), jnp.float32),
        ],
        scratch_types=scratch,
        compiler_params=pltpu.CompilerParams(use_tc_tiling_on_sc=False),
    )
    def ker(src0_hbm, src1_hbm, edge_hbm, sidx_hbm, didx_hbm, z64_hbm,
            z16_hbm, psum_hbm, pdeg_hbm, *scr):
        sidx = scr[0:NBUF]
        didx = scr[NBUF:2 * NBUF]
        srows = scr[2 * NBUF:3 * NBUF]
        erows = scr[3 * NBUF:4 * NBUF]
        ones_v, ssum, sdeg = scr[4 * NBUF:4 * NBUF + 3]
        sem_i = scr[4 * NBUF + 3:5 * NBUF + 3]
        sem_g = scr[5 * NBUF + 3:6 * NBUF + 3]
        sem_s = scr[6 * NBUF + 3:7 * NBUF + 3]

        cid = lax.axis_index("c")
        sid = lax.axis_index("s")

        onevec = jnp.where(lax.iota(jnp.int32, 16) == 0, 1.0, 0.0)

        def ones_body(j, _):
            ones_v[j, pl.ds(0, 16)] = onevec
            return 0

        lax.fori_loop(0, CHUNK, ones_body, 0)

        row0 = sid * ROWS_PER_TILE
        pltpu.sync_copy(z64_hbm, ssum.at[pl.ds(row0, ROWS_PER_TILE)])
        pltpu.sync_copy(z16_hbm, sdeg.at[pl.ds(row0, ROWS_PER_TILE)])

        tail0 = NUM_SUBCORES * ROWS_PER_TILE

        @pl.when(sid == NUM_SUBCORES - 1)
        def _zero_tail():
            pltpu.sync_copy(z64_hbm.at[pl.ds(0, TAIL_ROWS)],
                            ssum.at[pl.ds(tail0, TAIL_ROWS)])
            pltpu.sync_copy(z16_hbm.at[pl.ds(0, TAIL_ROWS)],
                            sdeg.at[pl.ds(tail0, TAIL_ROWS)])

        plsc.subcore_barrier()

        def issue_idx(c, b):
            base = sid * E_PER_TILE + c * CHUNK
            pltpu.async_copy(sidx_hbm.at[pl.ds(base, CHUNK)], sidx[b].at[0],
                             sem_i[b])
            pltpu.async_copy(didx_hbm.at[pl.ds(base, CHUNK)], didx[b].at[0],
                             sem_i[b])

        def wait_idx(c, b):
            base = sid * E_PER_TILE + c * CHUNK
            pltpu.make_async_copy(sidx_hbm.at[pl.ds(base, CHUNK)],
                                  sidx[b].at[0], sem_i[b]).wait()
            pltpu.make_async_copy(didx_hbm.at[pl.ds(base, CHUNK)],
                                  didx[b].at[0], sem_i[b]).wait()

        def issue_loads(c, b):
            idx = sidx[b].at[0]

            @pl.when(cid == 0)
            def _g0():
                pltpu.async_copy(src0_hbm.at[idx], srows[b], sem_g[b])
                pltpu.async_copy(
                    edge_hbm.at[pl.ds(sid * E_PER_TILE + c * CHUNK, CHUNK), 0],
                    erows[b], sem_g[b])

            @pl.when(cid == 1)
            def _g1():
                pltpu.async_copy(src1_hbm.at[idx], srows[b], sem_g[b])
                pltpu.async_copy(
                    edge_hbm.at[pl.ds(sid * E_PER_TILE + c * CHUNK, CHUNK), 1],
                    erows[b], sem_g[b])

        def wait_loads(c, b):
            pltpu.make_async_copy(src0_hbm.at[sidx[b].at[0]], srows[b],
                                  sem_g[b]).wait()
            pltpu.make_async_copy(
                edge_hbm.at[pl.ds(sid * E_PER_TILE + c * CHUNK, CHUNK), 0],
                erows[b], sem_g[b]).wait()

        def issue_scatters(c, b):
            idx = didx[b].at[0]
            pltpu.async_copy(srows[b], ssum.at[idx], sem_s[b], add=True)
            pltpu.async_copy(erows[b], ssum.at[idx], sem_s[b], add=True)
            pltpu.async_copy(ones_v, sdeg.at[idx], sem_s[b], add=True)

        def wait_scatters(c, b):
            idx = didx[b].at[0]
            pltpu.make_async_copy(srows[b], ssum.at[idx], sem_s[b]).wait()
            pltpu.make_async_copy(erows[b], ssum.at[idx], sem_s[b]).wait()
            pltpu.make_async_copy(ones_v, sdeg.at[idx], sem_s[b]).wait()

        for c in range(LA):
            issue_idx(c, c % NBUF)
        for c in range(LA):
            wait_idx(c, c % NBUF)
            issue_loads(c, c % NBUF)
        issue_idx(LA, LA % NBUF)

        def ring_body(g, _):
            for b in range(NBUF):
                c = g * NBUF + b
                wait_loads(c, b)
                issue_scatters(c, b)

                c3 = c + LA + 1
                b3 = (b + LA + 1) % NBUF

                @pl.when(c3 < N_CHUNKS)
                def _idx_prefetch():
                    issue_idx(c3, b3)

                c2 = c + LA
                b2 = (b + LA) % NBUF

                @pl.when(c2 < N_CHUNKS)
                def _load_prefetch():
                    @pl.when(c >= NBUF - LA)
                    def _drain_prev():
                        wait_scatters(c2, b2)

                    wait_idx(c2, b2)
                    issue_loads(c2, b2)

            return 0

        lax.fori_loop(0, N_CHUNKS // NBUF, ring_body, 0)

        for k in range(NBUF):
            cw = N_CHUNKS - NBUF + k
            wait_scatters(cw, cw % NBUF)

        plsc.subcore_barrier()

        pltpu.sync_copy(ssum.at[pl.ds(row0, ROWS_PER_TILE)],
                        psum_hbm.at[cid, pl.ds(row0, ROWS_PER_TILE)])
        pltpu.sync_copy(sdeg.at[pl.ds(row0, ROWS_PER_TILE)],
                        pdeg_hbm.at[cid, pl.ds(row0, ROWS_PER_TILE)])

        @pl.when(sid == NUM_SUBCORES - 1)
        def _write_tail():
            pltpu.sync_copy(ssum.at[pl.ds(tail0, TAIL_ROWS)],
                            psum_hbm.at[cid, pl.ds(tail0, TAIL_ROWS)])
            pltpu.sync_copy(sdeg.at[pl.ds(tail0, TAIL_ROWS)],
                            pdeg_hbm.at[cid, pl.ds(tail0, TAIL_ROWS)])

    return ker(src0, src1, edge3, src_idx, dst_idx, z64, z16)


def _combine_body(psum_ref, pdeg_ref, out_ref):
    s = jnp.concatenate([psum_ref[0], psum_ref[1]], axis=-1)
    d = pdeg_ref[0, :, :1]
    out_ref[...] = s / jnp.maximum(d, 1.0)


_RB = 1000


def _combine(psum, pdeg):
    return pl.pallas_call(
        _combine_body,
        grid=(N_NODES // _RB,),
        in_specs=[
            pl.BlockSpec((NUM_CORES, _RB, DH), lambda i: (0, i, 0)),
            pl.BlockSpec((NUM_CORES, _RB, 16), lambda i: (0, i, 0)),
        ],
        out_specs=pl.BlockSpec((_RB, D), lambda i: (i, 0)),
        out_shape=jax.ShapeDtypeStruct((N_NODES, D), jnp.float32),
    )(psum, pdeg)


@jax.jit
def kernel(src_embedding, edge_embedding, edge_index):
    src_idx = edge_index[0].astype(jnp.int32)
    dst_idx = edge_index[1].astype(jnp.int32)
    src0 = src_embedding[:, :DH]
    src1 = src_embedding[:, DH:]
    edge3 = edge_embedding.reshape(N_EDGES, NUM_CORES, DH)
    z64 = jnp.zeros((ROWS_PER_TILE, DH), jnp.float32)
    z16 = jnp.zeros((ROWS_PER_TILE, 16), jnp.float32)
    psum, pdeg = _sc_agg(src0, src1, edge3, src_idx, dst_idx, z64, z16)
    return _combine(psum, pdeg)


# speedup vs baseline: 8.3018x; 4.3266x over previous
import functools

import jax
import jax.numpy as jnp
from jax import lax
from jax.experimental import pallas as pl
from jax.experimental.pallas import tpu as pltpu
from jax.experimental.pallas import tpu_sc as plsc

N_NODES = 10000
N_EDGES = 320000
D = 128
DH = D // 2

NUM_CORES = 2
NUM_SUBCORES = 16
E_PER_TILE = N_EDGES // NUM_SUBCORES
CHUNK = 40
N_CHUNKS = E_PER_TILE // CHUNK
NBUF = 5
LA = 3
ROWS_PER_TILE = 624
TAIL_ROWS = N_NODES - NUM_SUBCORES * ROWS_PER_TILE


def _sc_agg(src0, src1, edge, src_idx, dst_idx, z64, z16):
    mesh = plsc.VectorSubcoreMesh(core_axis_name="c", subcore_axis_name="s")

    scratch = (
        [pltpu.VMEM((1, CHUNK), jnp.int32)] * NBUF
        + [pltpu.VMEM((1, CHUNK), jnp.int32)] * NBUF
        + [pltpu.VMEM((CHUNK, DH), jnp.float32)] * NBUF
        + [pltpu.VMEM((CHUNK, DH), jnp.float32)] * NBUF
        + [
            pltpu.VMEM((CHUNK, 16), jnp.float32),
            pltpu.VMEM_SHARED((N_NODES, DH), jnp.float32),
            pltpu.VMEM_SHARED((N_NODES, 16), jnp.float32),
        ]
        + [pltpu.SemaphoreType.DMA] * (3 * NBUF)
    )

    @functools.partial(
        pl.kernel,
        mesh=mesh,
        out_type=[
            jax.ShapeDtypeStruct((NUM_CORES, N_NODES, DH), jnp.float32),
            jax.ShapeDtypeStruct((NUM_CORES, N_NODES, 16), jnp.float32),
        ],
        scratch_types=scratch,
        compiler_params=pltpu.CompilerParams(use_tc_tiling_on_sc=False),
    )
    def ker(src0_hbm, src1_hbm, edge_hbm, sidx_hbm, didx_hbm, z64_hbm,
            z16_hbm, psum_hbm, pdeg_hbm, *scr):
        sidx = scr[0:NBUF]
        didx = scr[NBUF:2 * NBUF]
        srows = scr[2 * NBUF:3 * NBUF]
        erows = scr[3 * NBUF:4 * NBUF]
        ones_v, ssum, sdeg = scr[4 * NBUF:4 * NBUF + 3]
        sem_i = scr[4 * NBUF + 3:5 * NBUF + 3]
        sem_g = scr[5 * NBUF + 3:6 * NBUF + 3]
        sem_s = scr[6 * NBUF + 3:7 * NBUF + 3]

        cid = lax.axis_index("c")
        sid = lax.axis_index("s")

        onevec = jnp.where(lax.iota(jnp.int32, 16) == 0, 1.0, 0.0)

        def ones_body(j, _):
            ones_v[j, pl.ds(0, 16)] = onevec
            return 0

        lax.fori_loop(0, CHUNK, ones_body, 0)

        row0 = sid * ROWS_PER_TILE
        pltpu.sync_copy(z64_hbm, ssum.at[pl.ds(row0, ROWS_PER_TILE)])
        pltpu.sync_copy(z16_hbm, sdeg.at[pl.ds(row0, ROWS_PER_TILE)])

        tail0 = NUM_SUBCORES * ROWS_PER_TILE

        @pl.when(sid == NUM_SUBCORES - 1)
        def _zero_tail():
            pltpu.sync_copy(z64_hbm.at[pl.ds(0, TAIL_ROWS)],
                            ssum.at[pl.ds(tail0, TAIL_ROWS)])
            pltpu.sync_copy(z16_hbm.at[pl.ds(0, TAIL_ROWS)],
                            sdeg.at[pl.ds(tail0, TAIL_ROWS)])

        plsc.subcore_barrier()

        def issue_idx(c, b):
            base = sid * E_PER_TILE + c * CHUNK
            pltpu.async_copy(sidx_hbm.at[pl.ds(base, CHUNK)], sidx[b].at[0],
                             sem_i[b])
            pltpu.async_copy(didx_hbm.at[pl.ds(base, CHUNK)], didx[b].at[0],
                             sem_i[b])

        def wait_idx(c, b):
            base = sid * E_PER_TILE + c * CHUNK
            pltpu.make_async_copy(sidx_hbm.at[pl.ds(base, CHUNK)],
                                  sidx[b].at[0], sem_i[b]).wait()
            pltpu.make_async_copy(didx_hbm.at[pl.ds(base, CHUNK)],
                                  didx[b].at[0], sem_i[b]).wait()

        def issue_loads(c, b):
            idx = sidx[b].at[0]

            row_sl = pl.ds(sid * E_PER_TILE + c * CHUNK, CHUNK)

            @pl.when(cid == 0)
            def _g0():
                pltpu.async_copy(src0_hbm.at[idx], srows[b], sem_g[b])
                pltpu.async_copy(edge_hbm.at[row_sl, pl.ds(0, DH)],
                                 erows[b], sem_g[b])

            @pl.when(cid == 1)
            def _g1():
                pltpu.async_copy(src1_hbm.at[idx], srows[b], sem_g[b])
                pltpu.async_copy(edge_hbm.at[row_sl, pl.ds(DH, DH)],
                                 erows[b], sem_g[b])

        def wait_loads(c, b):
            pltpu.make_async_copy(src0_hbm.at[sidx[b].at[0]], srows[b],
                                  sem_g[b]).wait()
            pltpu.make_async_copy(
                edge_hbm.at[pl.ds(sid * E_PER_TILE + c * CHUNK, CHUNK),
                            pl.ds(0, DH)],
                erows[b], sem_g[b]).wait()

        def issue_scatters(c, b):
            idx = didx[b].at[0]
            pltpu.async_copy(srows[b], ssum.at[idx], sem_s[b], add=True)
            pltpu.async_copy(erows[b], ssum.at[idx], sem_s[b], add=True)
            pltpu.async_copy(ones_v, sdeg.at[idx], sem_s[b], add=True)

        def wait_scatters(c, b):
            idx = didx[b].at[0]
            pltpu.make_async_copy(srows[b], ssum.at[idx], sem_s[b]).wait()
            pltpu.make_async_copy(erows[b], ssum.at[idx], sem_s[b]).wait()
            pltpu.make_async_copy(ones_v, sdeg.at[idx], sem_s[b]).wait()

        for c in range(LA):
            issue_idx(c, c % NBUF)
        for c in range(LA):
            wait_idx(c, c % NBUF)
            issue_loads(c, c % NBUF)
        issue_idx(LA, LA % NBUF)

        def ring_body(g, _):
            for b in range(NBUF):
                c = g * NBUF + b
                wait_loads(c, b)
                issue_scatters(c, b)

                c3 = c + LA + 1
                b3 = (b + LA + 1) % NBUF

                @pl.when(c3 < N_CHUNKS)
                def _idx_prefetch():
                    issue_idx(c3, b3)

                c2 = c + LA
                b2 = (b + LA) % NBUF

                @pl.when(c2 < N_CHUNKS)
                def _load_prefetch():
                    @pl.when(c >= NBUF - LA)
                    def _drain_prev():
                        wait_scatters(c2, b2)

                    wait_idx(c2, b2)
                    issue_loads(c2, b2)

            return 0

        lax.fori_loop(0, N_CHUNKS // NBUF, ring_body, 0)

        for k in range(NBUF):
            cw = N_CHUNKS - NBUF + k
            wait_scatters(cw, cw % NBUF)

        plsc.subcore_barrier()

        pltpu.sync_copy(ssum.at[pl.ds(row0, ROWS_PER_TILE)],
                        psum_hbm.at[cid, pl.ds(row0, ROWS_PER_TILE)])
        pltpu.sync_copy(sdeg.at[pl.ds(row0, ROWS_PER_TILE)],
                        pdeg_hbm.at[cid, pl.ds(row0, ROWS_PER_TILE)])

        @pl.when(sid == NUM_SUBCORES - 1)
        def _write_tail():
            pltpu.sync_copy(ssum.at[pl.ds(tail0, TAIL_ROWS)],
                            psum_hbm.at[cid, pl.ds(tail0, TAIL_ROWS)])
            pltpu.sync_copy(sdeg.at[pl.ds(tail0, TAIL_ROWS)],
                            pdeg_hbm.at[cid, pl.ds(tail0, TAIL_ROWS)])

    return ker(src0, src1, edge, src_idx, dst_idx, z64, z16)


def _combine_body(psum_ref, pdeg_ref, out_ref):
    s = jnp.concatenate([psum_ref[0], psum_ref[1]], axis=-1)
    d = pdeg_ref[0, :, :1]
    out_ref[...] = s / jnp.maximum(d, 1.0)


_RB = 1000


def _combine(psum, pdeg):
    return pl.pallas_call(
        _combine_body,
        grid=(N_NODES // _RB,),
        in_specs=[
            pl.BlockSpec((NUM_CORES, _RB, DH), lambda i: (0, i, 0)),
            pl.BlockSpec((NUM_CORES, _RB, 16), lambda i: (0, i, 0)),
        ],
        out_specs=pl.BlockSpec((_RB, D), lambda i: (i, 0)),
        out_shape=jax.ShapeDtypeStruct((N_NODES, D), jnp.float32),
    )(psum, pdeg)


@jax.jit
def kernel(src_embedding, edge_embedding, edge_index):
    src_idx = edge_index[0].astype(jnp.int32)
    dst_idx = edge_index[1].astype(jnp.int32)
    src0 = src_embedding[:, :DH]
    src1 = src_embedding[:, DH:]
    z64 = jnp.zeros((ROWS_PER_TILE, DH), jnp.float32)
    z16 = jnp.zeros((ROWS_PER_TILE, 16), jnp.float32)
    psum, pdeg = _sc_agg(src0, src1, edge_embedding, src_idx, dst_idx,
                         z64, z16)
    return _combine(psum, pdeg)


# speedup vs baseline: 8.3418x; 1.0048x over previous
import functools

import jax
import jax.numpy as jnp
from jax import lax
from jax.experimental import pallas as pl
from jax.experimental.pallas import tpu as pltpu
from jax.experimental.pallas import tpu_sc as plsc

N_NODES = 10000
N_EDGES = 320000
D = 128
DH = D // 2

NUM_CORES = 2
NUM_SUBCORES = 16
E_PER_TILE = N_EDGES // NUM_SUBCORES
CHUNK = 40
N_CHUNKS = E_PER_TILE // CHUNK
NBUF = 5
LA = 3
ROWS_PER_TILE = 624
TAIL_ROWS = N_NODES - NUM_SUBCORES * ROWS_PER_TILE


def _sc_agg(src0, src1, edge, src_idx, dst_idx, z64, z16):
    mesh = plsc.VectorSubcoreMesh(core_axis_name="c", subcore_axis_name="s")

    scratch = (
        [pltpu.VMEM((1, CHUNK), jnp.int32)] * NBUF
        + [pltpu.VMEM((1, CHUNK), jnp.int32)] * NBUF
        + [pltpu.VMEM((CHUNK, DH), jnp.float32)] * NBUF
        + [pltpu.VMEM((CHUNK, DH), jnp.float32)] * NBUF
        + [
            pltpu.VMEM((CHUNK, 16), jnp.float32),
            pltpu.VMEM_SHARED((N_NODES, DH), jnp.float32),
            pltpu.VMEM_SHARED((N_NODES, 16), jnp.float32),
        ]
        + [pltpu.SemaphoreType.DMA] * (3 * NBUF)
    )

    @functools.partial(
        pl.kernel,
        mesh=mesh,
        out_type=[
            jax.ShapeDtypeStruct((NUM_CORES, N_NODES, DH), jnp.float32),
            jax.ShapeDtypeStruct((NUM_CORES, N_NODES, 16), jnp.float32),
        ],
        scratch_types=scratch,
        compiler_params=pltpu.CompilerParams(use_tc_tiling_on_sc=False),
    )
    def ker(src0_hbm, src1_hbm, edge_hbm, sidx_hbm, didx_hbm, z64_hbm,
            z16_hbm, psum_hbm, pdeg_hbm, *scr):
        sidx = scr[0:NBUF]
        didx = scr[NBUF:2 * NBUF]
        srows = scr[2 * NBUF:3 * NBUF]
        erows = scr[3 * NBUF:4 * NBUF]
        ones_v, ssum, sdeg = scr[4 * NBUF:4 * NBUF + 3]
        sem_i = scr[4 * NBUF + 3:5 * NBUF + 3]
        sem_g = scr[5 * NBUF + 3:6 * NBUF + 3]
        sem_s = scr[6 * NBUF + 3:7 * NBUF + 3]

        cid = lax.axis_index("c")
        sid = lax.axis_index("s")

        onevec = jnp.where(lax.iota(jnp.int32, 16) == 0, 1.0, 0.0)

        def ones_body(j, _):
            ones_v[j, pl.ds(0, 16)] = onevec
            return 0

        lax.fori_loop(0, CHUNK, ones_body, 0)

        row0 = sid * ROWS_PER_TILE
        pltpu.sync_copy(z64_hbm, ssum.at[pl.ds(row0, ROWS_PER_TILE)])
        pltpu.sync_copy(z16_hbm, sdeg.at[pl.ds(row0, ROWS_PER_TILE)])

        tail0 = NUM_SUBCORES * ROWS_PER_TILE

        @pl.when(sid == NUM_SUBCORES - 1)
        def _zero_tail():
            pltpu.sync_copy(z64_hbm.at[pl.ds(0, TAIL_ROWS)],
                            ssum.at[pl.ds(tail0, TAIL_ROWS)])
            pltpu.sync_copy(z16_hbm.at[pl.ds(0, TAIL_ROWS)],
                            sdeg.at[pl.ds(tail0, TAIL_ROWS)])

        plsc.subcore_barrier()

        def issue_idx(c, b):
            base = sid * E_PER_TILE + c * CHUNK
            pltpu.async_copy(sidx_hbm.at[pl.ds(base, CHUNK)], sidx[b].at[0],
                             sem_i[b])
            pltpu.async_copy(didx_hbm.at[pl.ds(base, CHUNK)], didx[b].at[0],
                             sem_i[b])

        def wait_idx(c, b):
            base = sid * E_PER_TILE + c * CHUNK
            pltpu.make_async_copy(sidx_hbm.at[pl.ds(base, CHUNK)],
                                  sidx[b].at[0], sem_i[b]).wait()
            pltpu.make_async_copy(didx_hbm.at[pl.ds(base, CHUNK)],
                                  didx[b].at[0], sem_i[b]).wait()

        def issue_loads(c, b):
            idx = sidx[b].at[0]

            row_sl = pl.ds(sid * E_PER_TILE + c * CHUNK, CHUNK)

            @pl.when(cid == 0)
            def _g0():
                pltpu.async_copy(src0_hbm.at[idx], srows[b], sem_g[b])
                pltpu.async_copy(edge_hbm.at[row_sl, pl.ds(0, DH)],
                                 erows[b], sem_g[b])

            @pl.when(cid == 1)
            def _g1():
                pltpu.async_copy(src1_hbm.at[idx], srows[b], sem_g[b])
                pltpu.async_copy(edge_hbm.at[row_sl, pl.ds(DH, DH)],
                                 erows[b], sem_g[b])

        def wait_loads(c, b):
            pltpu.make_async_copy(src0_hbm.at[sidx[b].at[0]], srows[b],
                                  sem_g[b]).wait()
            pltpu.make_async_copy(
                edge_hbm.at[pl.ds(sid * E_PER_TILE + c * CHUNK, CHUNK),
                            pl.ds(0, DH)],
                erows[b], sem_g[b]).wait()

        def deg_cond(c):
            return (c < N_CHUNKS // 2) == (cid == 0)

        def issue_scatters(c, b):
            idx = didx[b].at[0]
            pltpu.async_copy(srows[b], ssum.at[idx], sem_s[b], add=True)
            pltpu.async_copy(erows[b], ssum.at[idx], sem_s[b], add=True)

            @pl.when(deg_cond(c))
            def _deg():
                pltpu.async_copy(ones_v, sdeg.at[idx], sem_s[b], add=True)

        def wait_scatters(c, b):
            idx = didx[b].at[0]
            pltpu.make_async_copy(srows[b], ssum.at[idx], sem_s[b]).wait()
            pltpu.make_async_copy(erows[b], ssum.at[idx], sem_s[b]).wait()

            @pl.when(deg_cond(c))
            def _deg():
                pltpu.make_async_copy(ones_v, sdeg.at[idx], sem_s[b]).wait()

        for c in range(LA):
            issue_idx(c, c % NBUF)
        for c in range(LA):
            wait_idx(c, c % NBUF)
            issue_loads(c, c % NBUF)
        issue_idx(LA, LA % NBUF)

        def ring_body(g, _):
            for b in range(NBUF):
                c = g * NBUF + b
                wait_loads(c, b)
                issue_scatters(c, b)

                c3 = c + LA + 1
                b3 = (b + LA + 1) % NBUF

                @pl.when(c3 < N_CHUNKS)
                def _idx_prefetch():
                    issue_idx(c3, b3)

                c2 = c + LA
                b2 = (b + LA) % NBUF

                @pl.when(c2 < N_CHUNKS)
                def _load_prefetch():
                    @pl.when(c >= NBUF - LA)
                    def _drain_prev():
                        wait_scatters(c - (NBUF - LA), b2)

                    wait_idx(c2, b2)
                    issue_loads(c2, b2)

            return 0

        lax.fori_loop(0, N_CHUNKS // NBUF, ring_body, 0)

        for k in range(NBUF):
            cw = N_CHUNKS - NBUF + k
            wait_scatters(cw, cw % NBUF)

        plsc.subcore_barrier()

        pltpu.sync_copy(ssum.at[pl.ds(row0, ROWS_PER_TILE)],
                        psum_hbm.at[cid, pl.ds(row0, ROWS_PER_TILE)])
        pltpu.sync_copy(sdeg.at[pl.ds(row0, ROWS_PER_TILE)],
                        pdeg_hbm.at[cid, pl.ds(row0, ROWS_PER_TILE)])

        @pl.when(sid == NUM_SUBCORES - 1)
        def _write_tail():
            pltpu.sync_copy(ssum.at[pl.ds(tail0, TAIL_ROWS)],
                            psum_hbm.at[cid, pl.ds(tail0, TAIL_ROWS)])
            pltpu.sync_copy(sdeg.at[pl.ds(tail0, TAIL_ROWS)],
                            pdeg_hbm.at[cid, pl.ds(tail0, TAIL_ROWS)])

    return ker(src0, src1, edge, src_idx, dst_idx, z64, z16)


def _combine_body(psum_ref, pdeg_ref, out_ref):
    s = jnp.concatenate([psum_ref[0], psum_ref[1]], axis=-1)
    d = pdeg_ref[0, :, :1] + pdeg_ref[1, :, :1]
    out_ref[...] = s / jnp.maximum(d, 1.0)


_RB = 1000


def _combine(psum, pdeg):
    return pl.pallas_call(
        _combine_body,
        grid=(N_NODES // _RB,),
        in_specs=[
            pl.BlockSpec((NUM_CORES, _RB, DH), lambda i: (0, i, 0)),
            pl.BlockSpec((NUM_CORES, _RB, 16), lambda i: (0, i, 0)),
        ],
        out_specs=pl.BlockSpec((_RB, D), lambda i: (i, 0)),
        out_shape=jax.ShapeDtypeStruct((N_NODES, D), jnp.float32),
    )(psum, pdeg)


@jax.jit
def kernel(src_embedding, edge_embedding, edge_index):
    src_idx = edge_index[0].astype(jnp.int32)
    dst_idx = edge_index[1].astype(jnp.int32)
    src0 = src_embedding[:, :DH]
    src1 = src_embedding[:, DH:]
    z64 = jnp.zeros((ROWS_PER_TILE, DH), jnp.float32)
    z16 = jnp.zeros((ROWS_PER_TILE, 16), jnp.float32)
    psum, pdeg = _sc_agg(src0, src1, edge_embedding, src_idx, dst_idx,
                         z64, z16)
    return _combine(psum, pdeg)
